# Initial kernel scaffold; baseline (speedup 1.0000x reference)
import jax
import jax.numpy as jnp
from jax.experimental import pallas as pl


def kernel(attack_logits):
    raise NotImplementedError("write your pallas kernel here")


# speedup vs baseline: 1.9645x; 1.9645x over previous
import jax
import jax.numpy as jnp
from jax.experimental import pallas as pl


def _select_kernel(x_ref, out_ref):
    x = x_ref[:]
    idx = jnp.argmax(x, axis=-1)
    cols = jax.lax.broadcasted_iota(jnp.int32, x.shape, 1)
    out_ref[:] = (cols == idx[:, None]).astype(jnp.float32)


def kernel(attack_logits):
    b, n = attack_logits.shape
    return pl.pallas_call(
        _select_kernel,
        out_shape=jax.ShapeDtypeStruct((b, n), jnp.float32),
    )(attack_logits)
